# baseline (speedup 1.0000x reference)
import functools

import jax
import jax.numpy as jnp
from jax import lax
from jax.experimental import pallas as pl
from jax.experimental.pallas import tpu as pltpu
from jax.experimental.pallas import tpu_sc as plsc

_EPS = 1e-12

_NC = 2
_NS = 16
_NW = _NC * _NS
_CHUNK = 128


def _pick_chunk(rows_per_worker):
    for c in (128, 104, 96, 80, 64, 40, 32, 16, 8):
        if rows_per_worker % c == 0:
            return c
    raise ValueError(rows_per_worker)


def _sc_gather(table, idx_all, slice_idx):
    _, n_chunks, chunk = idx_all.shape
    d = table.shape[1]
    wbase = slice_idx * _NW
    mesh = plsc.VectorSubcoreMesh(core_axis_name="c", subcore_axis_name="s")

    nbuf = 5
    ring = n_chunks % nbuf == 0 and n_chunks >= 2 * nbuf

    @functools.partial(
        pl.kernel,
        mesh=mesh,
        out_type=jax.ShapeDtypeStruct((_NW, n_chunks, chunk, d), jnp.float32),
        scratch_types=[
            pltpu.VMEM((n_chunks, chunk), jnp.int32),
            pltpu.VMEM((nbuf, chunk, d), jnp.float32),
        ] + [pltpu.SemaphoreType.DMA] * (2 * nbuf),
    )
    def k(table_hbm, idx_hbm, out_hbm, idx_v, rows_v, *sems):
        gs = sems[:nbuf]
        ss = sems[nbuf:]
        wid = lax.axis_index("s") * _NC + lax.axis_index("c")
        pltpu.sync_copy(idx_hbm.at[wbase + wid], idx_v)

        def gather(c, u):
            pltpu.async_copy(table_hbm.at[idx_v.at[c]], rows_v.at[u], gs[u])

        def store(c, u):
            pltpu.async_copy(rows_v.at[u], out_hbm.at[wid, c], ss[u])

        def wait_g(u):
            pltpu.make_async_copy(out_hbm.at[wid, 0], rows_v.at[u], gs[u]).wait()

        def wait_s(u):
            pltpu.make_async_copy(out_hbm.at[wid, 0], rows_v.at[u], ss[u]).wait()

        if not ring:
            def body(c, carry):
                pltpu.async_copy(table_hbm.at[idx_v.at[c]], rows_v.at[0],
                                 gs[0]).wait()
                pltpu.sync_copy(rows_v.at[0], out_hbm.at[wid, c])
                return carry

            lax.fori_loop(0, n_chunks, body, 0)
            return

        n_t = n_chunks // nbuf
        for c in range(3):
            gather(c, c)
        for u in range(nbuf):
            wait_g(u)
            store(u, u)
            bp = (u + 3) % nbuf
            if u >= 2:
                wait_s(bp)
            gather(u + 3, bp)

        def body(t, carry):
            base = nbuf * t
            for u in range(nbuf):
                wait_g(u)
                store(base + u, u)
                bp = (u + 3) % nbuf
                wait_s(bp)
                gather(base + u + 3, bp)
            return carry

        lax.fori_loop(1, n_t - 1, body, 0)
        for u in range(nbuf):
            c = nbuf * (n_t - 1) + u
            wait_g(u)
            store(c, u)
            bp = (u + 3) % nbuf
            wait_s(bp)
            if c + 3 < n_chunks:
                gather(c + 3, bp)
        wait_s((n_chunks - 2) % nbuf)
        wait_s((n_chunks - 1) % nbuf)

    return k(table, idx_all).reshape(_NW * n_chunks * chunk, d)


def _ln(x, g, b):
    mean = jnp.mean(x, axis=-1, keepdims=True)
    xc = x - mean
    var = jnp.mean(xc * xc, axis=-1, keepdims=True)
    return xc * lax.rsqrt(var + _EPS) * g + b


def _tc_body(tg_ref, tt_ref, vis_ref, seg_ref, vseg_ref, g_ref, b_ref,
             vg_ref, vb_ref, out_ref):
    bb, seq_l, d = tg_ref.shape
    ttf = lax.broadcast_in_dim(tt_ref[...], (bb, seq_l, d), (0, 1))
    seg0 = seg_ref[0, :]
    dseg = seg_ref[1, :] - seg0
    text = tg_ref[...] + seg0 + ttf * dseg
    g = g_ref[...]
    b = b_ref[...]
    out_ref[:, :seq_l, :] = _ln(text, g, b)
    v = _ln(vis_ref[...], vg_ref[...], vb_ref[...]) + vseg_ref[0, :]
    out_ref[:, seq_l:, :] = _ln(v, g, b)


def _tc_body_alias(full_ref, tg_ref, tt_ref, vis_ref, seg_ref, vseg_ref,
                   g_ref, b_ref, vg_ref, vb_ref, out_ref):
    _tc_body(tg_ref, tt_ref, vis_ref, seg_ref, vseg_ref, g_ref, b_ref,
             vg_ref, vb_ref, out_ref)


def _tc_fuse_slice(full, gathered, token_type_ids, visual, seg_table,
                   v_seg_table, g, b, vg, vb, slice_idx, batch):
    bs, seq_l, d = gathered.shape
    f = visual.shape[1]
    bb = min(64, bs)
    grid = (bs // bb,)
    base = slice_idx * (bs // bb)
    in_specs = [
        pl.BlockSpec((bb, seq_l, d), lambda i: (i, 0, 0)),
        pl.BlockSpec((bb, seq_l), lambda i: (base + i, 0)),
        pl.BlockSpec((bb, f, d), lambda i: (base + i, 0, 0)),
        pl.BlockSpec((2, d), lambda i: (0, 0)),
        pl.BlockSpec((1, d), lambda i: (0, 0)),
        pl.BlockSpec((1, d), lambda i: (0, 0)),
        pl.BlockSpec((1, d), lambda i: (0, 0)),
        pl.BlockSpec((1, d), lambda i: (0, 0)),
        pl.BlockSpec((1, d), lambda i: (0, 0)),
    ]
    out_spec = pl.BlockSpec((bb, seq_l + f, d), lambda i: (base + i, 0, 0))
    out_shape = jax.ShapeDtypeStruct((batch, seq_l + f, d), jnp.float32)
    args = (gathered, token_type_ids, visual, seg_table, v_seg_table,
            g, b, vg, vb)
    if full is None:
        return pl.pallas_call(
            _tc_body, grid=grid, in_specs=in_specs,
            out_specs=out_spec, out_shape=out_shape,
        )(*args)
    full_spec = pl.BlockSpec((8, 8, d), lambda i: (0, 0, 0))
    return pl.pallas_call(
        _tc_body_alias, grid=grid, in_specs=[full_spec] + in_specs,
        out_specs=out_spec, out_shape=out_shape,
        input_output_aliases={0: 0},
    )(full, *args)


def kernel(input_ids, token_type_ids, input_mask, visual_embeds, visual_mask,
           tok_table, seg_table, v_seg_table, norm_gamma, norm_beta,
           vln_gamma, vln_beta):
    batch, seq_l = input_ids.shape
    d = tok_table.shape[1]
    n_slices = 4
    bs = batch // n_slices
    ids = input_ids.astype(jnp.int32)
    ttf = token_type_ids.astype(jnp.float32)
    g = norm_gamma.reshape(1, d)
    b = norm_beta.reshape(1, d)
    vg = vln_gamma.reshape(1, d)
    vb = vln_beta.reshape(1, d)
    rows_per_worker = bs * seq_l // _NW
    chunk = _pick_chunk(rows_per_worker)
    n_chunks = rows_per_worker // chunk
    idx_all = ids.reshape(n_slices * _NW, n_chunks, chunk)
    gathered = [
        _sc_gather(tok_table, idx_all, s).reshape(bs, seq_l, d)
        for s in range(n_slices)
    ]
    full = None
    for s in range(n_slices):
        full = _tc_fuse_slice(
            full, gathered[s], ttf, visual_embeds,
            seg_table, v_seg_table, g, b, vg, vb, s, batch,
        )
    out_mask = jnp.concatenate([input_mask, visual_mask], axis=1)
    return (full, out_mask)


# speedup vs baseline: 1.0558x; 1.0558x over previous
import functools

import jax
import jax.numpy as jnp
from jax import lax
from jax.experimental import pallas as pl
from jax.experimental.pallas import tpu as pltpu
from jax.experimental.pallas import tpu_sc as plsc

_EPS = 1e-12

_NC = 2
_NS = 16
_NW = _NC * _NS
_CHUNK = 128


def _pick_chunk(rows_per_worker):
    for c in (128, 104, 96, 80, 64, 40, 32, 16, 8):
        if rows_per_worker % c == 0:
            return c
    raise ValueError(rows_per_worker)


def _sc_gather(table, idx_all):
    _, n_chunks, chunk = idx_all.shape
    d = table.shape[1]
    mesh = plsc.VectorSubcoreMesh(core_axis_name="c", subcore_axis_name="s")

    nbuf = 5
    ring = n_chunks % nbuf == 0 and n_chunks >= 2 * nbuf

    @functools.partial(
        pl.kernel,
        mesh=mesh,
        out_type=jax.ShapeDtypeStruct((_NW, n_chunks, chunk, d), jnp.float32),
        scratch_types=[
            pltpu.VMEM((n_chunks, chunk), jnp.int32),
            pltpu.VMEM((nbuf, chunk, d), jnp.float32),
        ] + [pltpu.SemaphoreType.DMA] * (2 * nbuf),
    )
    def k(table_hbm, idx_hbm, out_hbm, idx_v, rows_v, *sems):
        gs = sems[:nbuf]
        ss = sems[nbuf:]
        wid = lax.axis_index("s") * _NC + lax.axis_index("c")
        pltpu.sync_copy(idx_hbm.at[wid], idx_v)

        def gather(c, u):
            pltpu.async_copy(table_hbm.at[idx_v.at[c]], rows_v.at[u], gs[u])

        def store(c, u):
            pltpu.async_copy(rows_v.at[u], out_hbm.at[wid, c], ss[u])

        def wait_g(u):
            pltpu.make_async_copy(out_hbm.at[wid, 0], rows_v.at[u], gs[u]).wait()

        def wait_s(u):
            pltpu.make_async_copy(out_hbm.at[wid, 0], rows_v.at[u], ss[u]).wait()

        if not ring:
            def body(c, carry):
                pltpu.async_copy(table_hbm.at[idx_v.at[c]], rows_v.at[0],
                                 gs[0]).wait()
                pltpu.sync_copy(rows_v.at[0], out_hbm.at[wid, c])
                return carry

            lax.fori_loop(0, n_chunks, body, 0)
            return

        n_t = n_chunks // nbuf
        for c in range(3):
            gather(c, c)
        for u in range(nbuf):
            wait_g(u)
            store(u, u)
            bp = (u + 3) % nbuf
            if u >= 2:
                wait_s(bp)
            gather(u + 3, bp)

        def body(t, carry):
            base = nbuf * t
            for u in range(nbuf):
                wait_g(u)
                store(base + u, u)
                bp = (u + 3) % nbuf
                wait_s(bp)
                gather(base + u + 3, bp)
            return carry

        lax.fori_loop(1, n_t - 1, body, 0)
        for u in range(nbuf):
            c = nbuf * (n_t - 1) + u
            wait_g(u)
            store(c, u)
            bp = (u + 3) % nbuf
            wait_s(bp)
            if c + 3 < n_chunks:
                gather(c + 3, bp)
        wait_s((n_chunks - 2) % nbuf)
        wait_s((n_chunks - 1) % nbuf)

    return k(table, idx_all).reshape(_NW * n_chunks * chunk, d)


def _ln(x, g, b):
    mean = jnp.mean(x, axis=-1, keepdims=True)
    xc = x - mean
    var = jnp.mean(xc * xc, axis=-1, keepdims=True)
    return xc * lax.rsqrt(var + _EPS) * g + b


def _tc_body(tg_ref, tt_ref, vis_ref, seg_ref, vseg_ref, g_ref, b_ref,
             vg_ref, vb_ref, out_ref):
    bb, seq_l, d = tg_ref.shape
    ttf = lax.broadcast_in_dim(tt_ref[...], (bb, seq_l, d), (0, 1))
    seg0 = seg_ref[0, :]
    dseg = seg_ref[1, :] - seg0
    text = tg_ref[...] + seg0 + ttf * dseg
    g = g_ref[...]
    b = b_ref[...]
    out_ref[:, :seq_l, :] = _ln(text, g, b)
    v = _ln(vis_ref[...], vg_ref[...], vb_ref[...]) + vseg_ref[0, :]
    out_ref[:, seq_l:, :] = _ln(v, g, b)


def _tc_body_alias(full_ref, tg_ref, tt_ref, vis_ref, seg_ref, vseg_ref,
                   g_ref, b_ref, vg_ref, vb_ref, out_ref):
    _tc_body(tg_ref, tt_ref, vis_ref, seg_ref, vseg_ref, g_ref, b_ref,
             vg_ref, vb_ref, out_ref)


def _tc_fuse_slice(full, gathered, token_type_ids, visual, seg_table,
                   v_seg_table, g, b, vg, vb, row0, batch):
    bs, seq_l, d = gathered.shape
    f = visual.shape[1]
    bb = min(64, bs)
    grid = (bs // bb,)
    base = row0 // bb
    in_specs = [
        pl.BlockSpec((bb, seq_l, d), lambda i: (i, 0, 0)),
        pl.BlockSpec((bb, seq_l), lambda i: (base + i, 0)),
        pl.BlockSpec((bb, f, d), lambda i: (base + i, 0, 0)),
        pl.BlockSpec((2, d), lambda i: (0, 0)),
        pl.BlockSpec((1, d), lambda i: (0, 0)),
        pl.BlockSpec((1, d), lambda i: (0, 0)),
        pl.BlockSpec((1, d), lambda i: (0, 0)),
        pl.BlockSpec((1, d), lambda i: (0, 0)),
        pl.BlockSpec((1, d), lambda i: (0, 0)),
    ]
    out_spec = pl.BlockSpec((bb, seq_l + f, d), lambda i: (base + i, 0, 0))
    out_shape = jax.ShapeDtypeStruct((batch, seq_l + f, d), jnp.float32)
    args = (gathered, token_type_ids, visual, seg_table, v_seg_table,
            g, b, vg, vb)
    if full is None:
        return pl.pallas_call(
            _tc_body, grid=grid, in_specs=in_specs,
            out_specs=out_spec, out_shape=out_shape,
        )(*args)
    full_spec = pl.BlockSpec((8, 8, d), lambda i: (0, 0, 0))
    return pl.pallas_call(
        _tc_body_alias, grid=grid, in_specs=[full_spec] + in_specs,
        out_specs=out_spec, out_shape=out_shape,
        input_output_aliases={0: 0},
    )(full, *args)


def kernel(input_ids, token_type_ids, input_mask, visual_embeds, visual_mask,
           tok_table, seg_table, v_seg_table, norm_gamma, norm_beta,
           vln_gamma, vln_beta):
    batch, seq_l = input_ids.shape
    d = tok_table.shape[1]
    sizes = (448, 576) if batch % 1024 == 0 else (batch // 2, batch - batch // 2)
    ids = input_ids.astype(jnp.int32)
    ttf = token_type_ids.astype(jnp.float32)
    g = norm_gamma.reshape(1, d)
    b = norm_beta.reshape(1, d)
    vg = vln_gamma.reshape(1, d)
    vb = vln_beta.reshape(1, d)
    ids_flat = ids.reshape(-1)
    gathered = []
    off = 0
    for bs in sizes:
        rows_per_worker = bs * seq_l // _NW
        chunk = _pick_chunk(rows_per_worker)
        n_chunks = rows_per_worker // chunk
        idx_s = ids_flat[off:off + bs * seq_l].reshape(_NW, n_chunks, chunk)
        gathered.append(_sc_gather(tok_table, idx_s).reshape(bs, seq_l, d))
        off += bs * seq_l
    full = None
    row0 = 0
    for s, bs in enumerate(sizes):
        full = _tc_fuse_slice(
            full, gathered[s], ttf, visual_embeds,
            seg_table, v_seg_table, g, b, vg, vb, row0, batch,
        )
        row0 += bs
    out_mask = jnp.concatenate([input_mask, visual_mask], axis=1)
    return (full, out_mask)


# speedup vs baseline: 1.0666x; 1.0102x over previous
import functools

import jax
import jax.numpy as jnp
from jax import lax
from jax.experimental import pallas as pl
from jax.experimental.pallas import tpu as pltpu
from jax.experimental.pallas import tpu_sc as plsc

_EPS = 1e-12

_NC = 2
_NS = 16
_NW = _NC * _NS
_CHUNK = 128


def _pick_chunk(rows_per_worker):
    for c in (128, 104, 96, 80, 64, 40, 32, 16, 8):
        if rows_per_worker % c == 0:
            return c
    raise ValueError(rows_per_worker)


def _sc_gather(table, idx_all):
    _, n_chunks, chunk = idx_all.shape
    d = table.shape[1]
    mesh = plsc.VectorSubcoreMesh(core_axis_name="c", subcore_axis_name="s")

    nbuf = 5
    ring = n_chunks % nbuf == 0 and n_chunks >= 2 * nbuf

    @functools.partial(
        pl.kernel,
        mesh=mesh,
        out_type=jax.ShapeDtypeStruct((_NW, n_chunks, chunk, d), jnp.float32),
        scratch_types=[
            pltpu.VMEM((n_chunks, chunk), jnp.int32),
            pltpu.VMEM((nbuf, chunk, d), jnp.float32),
        ] + [pltpu.SemaphoreType.DMA] * (2 * nbuf),
    )
    def k(table_hbm, idx_hbm, out_hbm, idx_v, rows_v, *sems):
        gs = sems[:nbuf]
        ss = sems[nbuf:]
        wid = lax.axis_index("s") * _NC + lax.axis_index("c")
        pltpu.sync_copy(idx_hbm.at[wid], idx_v)

        def gather(c, u):
            pltpu.async_copy(table_hbm.at[idx_v.at[c]], rows_v.at[u], gs[u])

        def store(c, u):
            pltpu.async_copy(rows_v.at[u], out_hbm.at[wid, c], ss[u])

        def wait_g(u):
            pltpu.make_async_copy(out_hbm.at[wid, 0], rows_v.at[u], gs[u]).wait()

        def wait_s(u):
            pltpu.make_async_copy(out_hbm.at[wid, 0], rows_v.at[u], ss[u]).wait()

        if not ring:
            def body(c, carry):
                pltpu.async_copy(table_hbm.at[idx_v.at[c]], rows_v.at[0],
                                 gs[0]).wait()
                pltpu.sync_copy(rows_v.at[0], out_hbm.at[wid, c])
                return carry

            lax.fori_loop(0, n_chunks, body, 0)
            return

        n_t = n_chunks // nbuf
        for c in range(3):
            gather(c, c)
        for u in range(nbuf):
            wait_g(u)
            store(u, u)
            bp = (u + 3) % nbuf
            if u >= 2:
                wait_s(bp)
            gather(u + 3, bp)

        def body(t, carry):
            base = nbuf * t
            for u in range(nbuf):
                wait_g(u)
                store(base + u, u)
                bp = (u + 3) % nbuf
                wait_s(bp)
                gather(base + u + 3, bp)
            return carry

        lax.fori_loop(1, n_t - 1, body, 0)
        for u in range(nbuf):
            c = nbuf * (n_t - 1) + u
            wait_g(u)
            store(c, u)
            bp = (u + 3) % nbuf
            wait_s(bp)
            if c + 3 < n_chunks:
                gather(c + 3, bp)
        wait_s((n_chunks - 2) % nbuf)
        wait_s((n_chunks - 1) % nbuf)

    return k(table, idx_all).reshape(_NW * n_chunks * chunk, d)


def _ln(x, g, b):
    mean = jnp.mean(x, axis=-1, keepdims=True)
    xc = x - mean
    var = jnp.mean(xc * xc, axis=-1, keepdims=True)
    return xc * lax.rsqrt(var + _EPS) * g + b


def _tc_body(tg_ref, tt_ref, vis_ref, seg_ref, vseg_ref, g_ref, b_ref,
             vg_ref, vb_ref, out_ref):
    bb, seq_l, d = tg_ref.shape
    ttf = lax.broadcast_in_dim(tt_ref[...], (bb, seq_l, d), (0, 1))
    seg0 = seg_ref[0, :]
    dseg = seg_ref[1, :] - seg0
    text = tg_ref[...] + seg0 + ttf * dseg
    g = g_ref[...]
    b = b_ref[...]
    out_ref[:, :seq_l, :] = _ln(text, g, b)
    v = _ln(vis_ref[...], vg_ref[...], vb_ref[...]) + vseg_ref[0, :]
    out_ref[:, seq_l:, :] = _ln(v, g, b)


def _tc_body_alias(full_ref, tg_ref, tt_ref, vis_ref, seg_ref, vseg_ref,
                   g_ref, b_ref, vg_ref, vb_ref, out_ref):
    _tc_body(tg_ref, tt_ref, vis_ref, seg_ref, vseg_ref, g_ref, b_ref,
             vg_ref, vb_ref, out_ref)


def _tc_fuse_slice(full, gathered, token_type_ids, visual, seg_table,
                   v_seg_table, g, b, vg, vb, row0, batch):
    bs, seq_l, d = gathered.shape
    f = visual.shape[1]
    bb = min(64, bs)
    grid = (bs // bb,)
    base = row0 // bb
    in_specs = [
        pl.BlockSpec((bb, seq_l, d), lambda i: (i, 0, 0)),
        pl.BlockSpec((bb, seq_l), lambda i: (base + i, 0)),
        pl.BlockSpec((bb, f, d), lambda i: (base + i, 0, 0)),
        pl.BlockSpec((2, d), lambda i: (0, 0)),
        pl.BlockSpec((1, d), lambda i: (0, 0)),
        pl.BlockSpec((1, d), lambda i: (0, 0)),
        pl.BlockSpec((1, d), lambda i: (0, 0)),
        pl.BlockSpec((1, d), lambda i: (0, 0)),
        pl.BlockSpec((1, d), lambda i: (0, 0)),
    ]
    out_spec = pl.BlockSpec((bb, seq_l + f, d), lambda i: (base + i, 0, 0))
    out_shape = jax.ShapeDtypeStruct((batch, seq_l + f, d), jnp.float32)
    args = (gathered, token_type_ids, visual, seg_table, v_seg_table,
            g, b, vg, vb)
    if full is None:
        return pl.pallas_call(
            _tc_body, grid=grid, in_specs=in_specs,
            out_specs=out_spec, out_shape=out_shape,
        )(*args)
    full_spec = pl.BlockSpec((8, 8, d), lambda i: (0, 0, 0))
    return pl.pallas_call(
        _tc_body_alias, grid=grid, in_specs=[full_spec] + in_specs,
        out_specs=out_spec, out_shape=out_shape,
        input_output_aliases={0: 0},
    )(full, *args)


def kernel(input_ids, token_type_ids, input_mask, visual_embeds, visual_mask,
           tok_table, seg_table, v_seg_table, norm_gamma, norm_beta,
           vln_gamma, vln_beta):
    batch, seq_l = input_ids.shape
    d = tok_table.shape[1]
    sizes = (256, 320, 448) if batch == 1024 else (batch // 2, batch - batch // 2)
    ids = input_ids.astype(jnp.int32)
    ttf = token_type_ids.astype(jnp.float32)
    g = norm_gamma.reshape(1, d)
    b = norm_beta.reshape(1, d)
    vg = vln_gamma.reshape(1, d)
    vb = vln_beta.reshape(1, d)
    ids_flat = ids.reshape(-1)
    gathered = []
    off = 0
    for bs in sizes:
        rows_per_worker = bs * seq_l // _NW
        chunk = _pick_chunk(rows_per_worker)
        n_chunks = rows_per_worker // chunk
        idx_s = ids_flat[off:off + bs * seq_l].reshape(_NW, n_chunks, chunk)
        gathered.append(_sc_gather(tok_table, idx_s).reshape(bs, seq_l, d))
        off += bs * seq_l
    full = None
    row0 = 0
    for s, bs in enumerate(sizes):
        full = _tc_fuse_slice(
            full, gathered[s], ttf, visual_embeds,
            seg_table, v_seg_table, g, b, vg, vb, row0, batch,
        )
        row0 += bs
    out_mask = jnp.concatenate([input_mask, visual_mask], axis=1)
    return (full, out_mask)
